# Initial kernel scaffold; baseline (speedup 1.0000x reference)
import jax
import jax.numpy as jnp
from jax.experimental import pallas as pl


def kernel(feat_plane, mask_indices):
    raise NotImplementedError("write your pallas kernel here")


# speedup vs baseline: 1.1168x; 1.1168x over previous
import functools

import jax
import jax.numpy as jnp
from jax import lax
from jax.experimental import pallas as pl
from jax.experimental.pallas import tpu as pltpu
from jax.experimental.pallas import tpu_sc as plsc

_NC, _NS = 2, 16
_NW = _NC * _NS
_CHUNK = 1024


@functools.lru_cache(maxsize=None)
def _make_gather(v_rows, d, b_rows):
    del v_rows
    assert b_rows % (_NW * _CHUNK) == 0
    b_per_w = b_rows // _NW
    n_chunks = b_per_w // _CHUNK
    mesh = plsc.VectorSubcoreMesh(core_axis_name="c", subcore_axis_name="s")

    @functools.partial(
        pl.kernel,
        out_type=jax.ShapeDtypeStruct((b_rows, d), jnp.float32),
        mesh=mesh,
        scratch_types=[
            pltpu.VMEM((b_per_w,), jnp.int32),
            pltpu.VMEM((2, _CHUNK, d), jnp.float32),
            pltpu.SemaphoreType.DMA,
            pltpu.SemaphoreType.DMA,
            pltpu.SemaphoreType.DMA,
            pltpu.SemaphoreType.DMA,
        ],
        compiler_params=pltpu.CompilerParams(use_tc_tiling_on_sc=False),
    )
    def gather_kernel(table_hbm, idx_hbm, out_hbm, idx_v, rows_v,
                      gsem0, gsem1, osem0, osem1):
        wid = lax.axis_index("s") * _NC + lax.axis_index("c")
        base = wid * b_per_w
        pltpu.sync_copy(idx_hbm.at[pl.ds(base, b_per_w)], idx_v)

        gsems = (gsem0, gsem1)
        osems = (osem0, osem1)
        gd = [None, None]
        od = [None, None]
        for i in range(n_chunks):
            bb = i & 1
            if i >= 2:
                od[bb].wait()
            gd[bb] = pltpu.async_copy(
                table_hbm.at[idx_v.at[pl.ds(i * _CHUNK, _CHUNK)]],
                rows_v.at[bb], gsems[bb])
            if i >= 1:
                gd[1 - bb].wait()
                od[1 - bb] = pltpu.async_copy(
                    rows_v.at[1 - bb],
                    out_hbm.at[pl.ds(base + (i - 1) * _CHUNK, _CHUNK)],
                    osems[1 - bb])
        last = n_chunks - 1
        bb = last & 1
        gd[bb].wait()
        od[bb] = pltpu.async_copy(
            rows_v.at[bb],
            out_hbm.at[pl.ds(base + last * _CHUNK, _CHUNK)],
            osems[bb])
        if n_chunks >= 2:
            od[1 - bb].wait()
        od[bb].wait()

    return gather_kernel


def kernel(feat_plane, mask_indices):
    d = feat_plane.shape[-1]
    flat = feat_plane.reshape(-1, d)
    idx = mask_indices.astype(jnp.int32)
    fn = _make_gather(flat.shape[0], d, idx.shape[0])
    return fn(flat, idx)
